# Initial kernel scaffold; baseline (speedup 1.0000x reference)
import jax
import jax.numpy as jnp
from jax.experimental import pallas as pl


def kernel(x, segment_ids, W1, b1, g1, be1, W2, b2, g2, be2, W3, b3, W4, b4, W5, b5, W6, b6):
    raise NotImplementedError("write your pallas kernel here")


# speedup vs baseline: 2.7684x; 2.7684x over previous
import functools

import jax
import jax.numpy as jnp
from jax import lax
from jax.experimental import pallas as pl
from jax.experimental.pallas import tpu as pltpu
from jax.experimental.pallas import tpu_sc as plsc

N = 50000
D = 512
H1 = 1536
H2 = 1024
NG = 2048
TN = 1000
NT = N // TN
D2 = 640

NW = 32
CHUNK = 1568
BR = 112
AROWS = 2176
TRASH = 2064
EPS = 1e-5


def _lrelu(h):
    return jnp.where(h > 0, h, 0.01 * h)


def _stage_a(x_ref, c_ref, sx_ref):
    i = pl.program_id(0)

    @pl.when(i == 0)
    def _():
        c_ref[...] = jnp.zeros_like(c_ref)
        sx_ref[...] = jnp.zeros_like(sx_ref)

    xb = x_ref[...].astype(jnp.bfloat16)
    c_ref[...] += lax.dot_general(xb, xb, (((0,), (0,)), ((), ())),
                                  preferred_element_type=jnp.float32)
    sx_ref[...] += jnp.sum(x_ref[...], axis=0, keepdims=True)


def _stage_p1(c_ref, sx_ref, w1_ref, b1_ref, g1_ref, be1_ref,
              w1p_ref, b1p_ref):
    w1 = w1_ref[...]
    w1b = w1.astype(jnp.bfloat16)
    cw = jnp.dot(c_ref[...].astype(jnp.bfloat16), w1b,
                 preferred_element_type=jnp.float32)
    q = jnp.sum(w1 * cw, axis=0, keepdims=True) / N
    mx = sx_ref[...] / N
    u = jnp.dot(mx.astype(jnp.bfloat16), w1b,
                preferred_element_type=jnp.float32)
    var = q - u * u
    scale = g1_ref[...] * lax.rsqrt(var + EPS)
    w1p_ref[...] = (w1 * scale).astype(jnp.bfloat16)
    b1p_ref[...] = be1_ref[...] - u * scale


def _stage_c(x_ref, w1p_ref, b1p_ref, w2_ref, b2_ref,
             h2_ref, s2_ref, s2sq_ref):
    i = pl.program_id(0)

    @pl.when(i == 0)
    def _():
        s2_ref[...] = jnp.zeros_like(s2_ref)
        s2sq_ref[...] = jnp.zeros_like(s2sq_ref)

    xb = x_ref[...].astype(jnp.bfloat16)
    h = jnp.dot(xb, w1p_ref[...], preferred_element_type=jnp.float32)
    h = _lrelu(h + b1p_ref[...])
    h2 = jnp.dot(h.astype(jnp.bfloat16), w2_ref[...],
                 preferred_element_type=jnp.float32) + b2_ref[...]
    h2_ref[...] = h2
    s2_ref[...] += jnp.sum(h2, axis=0, keepdims=True)
    s2sq_ref[...] += jnp.sum(h2 * h2, axis=0, keepdims=True)


def _stage_d(h2_ref, x_ref, s2_ref, s2sq_ref, g2_ref, be2_ref,
             w3_ref, b3_ref, w4_ref, b4_ref, w5_ref, b5_ref,
             w6_ref, b6_ref, xs2_ref):
    m2 = s2_ref[...] / N
    var2 = s2sq_ref[...] / N - m2 * m2
    scale2 = g2_ref[...] * lax.rsqrt(var2 + EPS)
    shift2 = be2_ref[...] - m2 * scale2
    h2 = _lrelu(h2_ref[...] * scale2 + shift2)
    h3 = _lrelu(jnp.dot(h2.astype(jnp.bfloat16), w3_ref[...],
                        preferred_element_type=jnp.float32) + b3_ref[...])
    h4 = _lrelu(jnp.dot(h3.astype(jnp.bfloat16), w4_ref[...],
                        preferred_element_type=jnp.float32) + b4_ref[...])
    h5 = _lrelu(jnp.dot(h4.astype(jnp.bfloat16), w5_ref[...],
                        preferred_element_type=jnp.float32) + b5_ref[...])
    logit = jnp.sum(h5 * w6_ref[...], axis=1, keepdims=True) + b6_ref[...]
    gate = jax.nn.sigmoid(logit)
    e = jnp.exp(gate)
    xe = x_ref[...] * e
    mask0 = lax.broadcasted_iota(jnp.int32, (TN, D2 - D), 1) == 0
    etail = jnp.where(mask0, e, 0.0)
    xs2_ref[...] = jnp.concatenate([xe, etail], axis=1)


def _sc_scatter(xs2_hbm, seg_hbm, sf_hbm, out_hbm, xbuf, segbuf, sfbuf):
    c = lax.axis_index("c")
    s = lax.axis_index("s")
    base_c = c * AROWS

    zrow = jnp.zeros((16,), jnp.float32)

    def zx(i, cr):
        xbuf[i // (D2 // 16), pl.ds((i % (D2 // 16)) * 16, 16)] = zrow
        return cr

    lax.fori_loop(0, BR * (D2 // 16), zx, 0)
    r0 = base_c + s * 136
    pltpu.sync_copy(xbuf, out_hbm.at[pl.ds(r0, BR)])
    pltpu.sync_copy(xbuf.at[pl.ds(0, 136 - BR)],
                    out_hbm.at[pl.ds(r0 + BR, 136 - BR)])
    plsc.subcore_barrier()

    pltpu.sync_copy(sf_hbm.at[pl.ds(c * 16, 16)], sfbuf)
    spl = jnp.take(sfbuf[...], jnp.full((16,), s, jnp.int32))

    w = c * 16 + s
    start = w * CHUNK
    cnt = jnp.minimum(N - start, CHUNK)
    nb = (cnt + BR - 1) // BR
    spill_row = base_c + NG + s
    trash_row = base_c + TRASH
    lanes16 = lax.iota(jnp.int32, 16)

    def blk(jb, cr):
        base = jnp.minimum(jb * BR, cnt - BR)
        dup = jb * BR - base
        rr = start + base
        pltpu.sync_copy(xs2_hbm.at[pl.ds(rr, BR)], xbuf)
        pltpu.sync_copy(seg_hbm.at[pl.ds(rr, BR)], segbuf)
        for kk in range(BR // 16):
            lane = lanes16 + (kk * 16)
            sgb = segbuf[pl.ds(kk * 16, 16)]
            red = jnp.where(sgb == spl, spill_row, sgb + base_c)
            red = jnp.where(lane < dup, trash_row, red)
            segbuf[pl.ds(kk * 16, 16)] = red
        pltpu.sync_copy(xbuf, out_hbm.at[segbuf], add=True)
        return cr

    lax.fori_loop(0, nb, blk, 0)


def _stage_f(encp_ref, sf_ref, out_ref):
    t = encp_ref[0:NG, :] + encp_ref[AROWS:AROWS + NG, :]
    sp0 = encp_ref[NG:NG + 16, :]
    sp1 = encp_ref[AROWS + NG:AROWS + NG + 16, :]
    spill = jnp.concatenate([sp0, sp1], axis=0)
    iota = lax.broadcasted_iota(jnp.int32, (NW, NG), 1).astype(jnp.float32)
    onehot = (sf_ref[...] == iota).astype(jnp.bfloat16)
    t = t + lax.dot_general(onehot, spill.astype(jnp.bfloat16),
                            (((0,), (0,)), ((), ())),
                            preferred_element_type=jnp.float32)
    enc = t[:, :D]
    den = t[:, D:D + 1]
    r = 1.0 / jnp.where(den == 0.0, 1.0, den)
    out_ref[...] = enc * r


def _tc_front(x, W1, b1, g1, be1, W2, b2, g2, be2,
              W3, b3, W4, b4, W5, b5, W6, b6):
    f32 = jnp.float32
    bf16 = jnp.bfloat16
    row = lambda v: v.reshape(1, -1).astype(f32)

    c_mat, sx = pl.pallas_call(
        _stage_a,
        grid=(NT,),
        in_specs=[pl.BlockSpec((TN, D), lambda i: (i, 0))],
        out_specs=[pl.BlockSpec((D, D), lambda i: (0, 0)),
                   pl.BlockSpec((1, D), lambda i: (0, 0))],
        out_shape=[jax.ShapeDtypeStruct((D, D), f32),
                   jax.ShapeDtypeStruct((1, D), f32)],
    )(x)

    w1p, b1p = pl.pallas_call(
        _stage_p1,
        out_shape=[jax.ShapeDtypeStruct((D, H1), bf16),
                   jax.ShapeDtypeStruct((1, H1), f32)],
    )(c_mat, sx, W1, row(b1), row(g1), row(be1))

    h2pre, s2, s2sq = pl.pallas_call(
        _stage_c,
        grid=(NT,),
        in_specs=[pl.BlockSpec((TN, D), lambda i: (i, 0)),
                  pl.BlockSpec((D, H1), lambda i: (0, 0)),
                  pl.BlockSpec((1, H1), lambda i: (0, 0)),
                  pl.BlockSpec((H1, H2), lambda i: (0, 0)),
                  pl.BlockSpec((1, H2), lambda i: (0, 0))],
        out_specs=[pl.BlockSpec((TN, H2), lambda i: (i, 0)),
                   pl.BlockSpec((1, H2), lambda i: (0, 0)),
                   pl.BlockSpec((1, H2), lambda i: (0, 0))],
        out_shape=[jax.ShapeDtypeStruct((N, H2), f32),
                   jax.ShapeDtypeStruct((1, H2), f32),
                   jax.ShapeDtypeStruct((1, H2), f32)],
    )(x, w1p, b1p, W2.astype(bf16), row(b2))

    xs2 = pl.pallas_call(
        _stage_d,
        grid=(NT,),
        in_specs=[pl.BlockSpec((TN, H2), lambda i: (i, 0)),
                  pl.BlockSpec((TN, D), lambda i: (i, 0)),
                  pl.BlockSpec((1, H2), lambda i: (0, 0)),
                  pl.BlockSpec((1, H2), lambda i: (0, 0)),
                  pl.BlockSpec((1, H2), lambda i: (0, 0)),
                  pl.BlockSpec((1, H2), lambda i: (0, 0)),
                  pl.BlockSpec((H2, 128), lambda i: (0, 0)),
                  pl.BlockSpec((1, 128), lambda i: (0, 0)),
                  pl.BlockSpec((128, 128), lambda i: (0, 0)),
                  pl.BlockSpec((1, 128), lambda i: (0, 0)),
                  pl.BlockSpec((128, 128), lambda i: (0, 0)),
                  pl.BlockSpec((1, 128), lambda i: (0, 0)),
                  pl.BlockSpec((1, 128), lambda i: (0, 0)),
                  pl.BlockSpec((1, 1), lambda i: (0, 0))],
        out_specs=[pl.BlockSpec((TN, D2), lambda i: (i, 0))],
        out_shape=[jax.ShapeDtypeStruct((N, D2), f32)],
    )(h2pre, x, s2, s2sq, row(g2), row(be2),
      W3.astype(bf16), row(b3), W4.astype(bf16), row(b4),
      W5.astype(bf16), row(b5), W6.reshape(1, 128).astype(f32),
      b6.reshape(1, 1).astype(f32))[0]
    return xs2


def _sc_call(xs2, seg, segfirst):
    f32 = jnp.float32
    mesh = plsc.VectorSubcoreMesh(core_axis_name="c", subcore_axis_name="s")
    encp = pl.kernel(
        _sc_scatter,
        out_type=jax.ShapeDtypeStruct((2 * AROWS, D2), f32),
        mesh=mesh,
        scratch_types=[pltpu.VMEM((BR, D2), f32),
                       pltpu.VMEM((BR,), jnp.int32),
                       pltpu.VMEM((16,), jnp.int32)],
    )(xs2, seg, segfirst)
    return encp


def _finalize(encp, sfcol):
    f32 = jnp.float32
    out = pl.pallas_call(
        _stage_f,
        in_specs=[pl.BlockSpec((2 * AROWS, D2), lambda: (0, 0)),
                  pl.BlockSpec((NW, 1), lambda: (0, 0))],
        out_specs=pl.BlockSpec((NG, D), lambda: (0, 0)),
        out_shape=jax.ShapeDtypeStruct((NG, D), f32),
    )(encp, sfcol)
    return out


def _stage_z(xs2_ref, segf_ref, out_ref):
    i = pl.program_id(0)

    @pl.when(i == 0)
    def _():
        out_ref[...] = jnp.zeros_like(out_ref)

    iota = lax.broadcasted_iota(jnp.int32, (TN, NG), 1).astype(jnp.float32)
    onehot = (segf_ref[...] == iota).astype(jnp.bfloat16)
    out_ref[...] += lax.dot_general(
        onehot, xs2_ref[...].astype(jnp.bfloat16),
        (((0,), (0,)), ((), ())), preferred_element_type=jnp.float32)


def _stage_zf(acc_ref, out_ref):
    t = acc_ref[...]
    enc = t[:, :D]
    den = t[:, D:D + 1]
    r = 1.0 / jnp.where(den == 0.0, 1.0, den)
    out_ref[...] = enc * r


def _tc_pool(xs2, segf):
    f32 = jnp.float32
    acc = pl.pallas_call(
        _stage_z,
        grid=(NT,),
        in_specs=[pl.BlockSpec((TN, D2), lambda i: (i, 0)),
                  pl.BlockSpec((TN, 1), lambda i: (i, 0))],
        out_specs=pl.BlockSpec((NG, D2), lambda i: (0, 0)),
        out_shape=jax.ShapeDtypeStruct((NG, D2), f32),
    )(xs2, segf)
    return pl.pallas_call(
        _stage_zf,
        in_specs=[pl.BlockSpec((NG, D2), lambda: (0, 0))],
        out_specs=pl.BlockSpec((NG, D), lambda: (0, 0)),
        out_shape=jax.ShapeDtypeStruct((NG, D), f32),
    )(acc)


def kernel(x, segment_ids, W1, b1, g1, be1, W2, b2, g2, be2,
           W3, b3, W4, b4, W5, b5, W6, b6):
    xs2 = _tc_front(x, W1, b1, g1, be1, W2, b2, g2, be2,
                    W3, b3, W4, b4, W5, b5, W6, b6)
    segf = segment_ids.astype(jnp.float32).reshape(N, 1)
    return _tc_pool(xs2, segf)


# speedup vs baseline: 2.8290x; 1.0219x over previous
import functools

import jax
import jax.numpy as jnp
from jax import lax
from jax.experimental import pallas as pl
from jax.experimental.pallas import tpu as pltpu
from jax.experimental.pallas import tpu_sc as plsc

N = 50000
D = 512
H1 = 1536
H2 = 1024
NG = 2048
TN = 1000
NT = N // TN
D2 = 640

NW = 32
CHUNK = 1568
BR = 112
AROWS = 2176
TRASH = 2064
EPS = 1e-5


def _lrelu(h):
    return jnp.where(h > 0, h, 0.01 * h)


def _stage_a(x_ref, c_ref, sx_ref):
    i = pl.program_id(0)

    @pl.when(i == 0)
    def _():
        c_ref[...] = jnp.zeros_like(c_ref)
        sx_ref[...] = jnp.zeros_like(sx_ref)

    xb = x_ref[...].astype(jnp.bfloat16)
    c_ref[...] += lax.dot_general(xb, xb, (((0,), (0,)), ((), ())),
                                  preferred_element_type=jnp.float32)
    sx_ref[...] += jnp.sum(x_ref[...], axis=0, keepdims=True)


def _stage_p1(c_ref, sx_ref, w1_ref, b1_ref, g1_ref, be1_ref,
              w1p_ref, b1p_ref):
    w1 = w1_ref[...]
    w1b = w1.astype(jnp.bfloat16)
    cw = jnp.dot(c_ref[...].astype(jnp.bfloat16), w1b,
                 preferred_element_type=jnp.float32)
    q = jnp.sum(w1 * cw, axis=0, keepdims=True) / N
    mx = sx_ref[...] / N
    u = jnp.dot(mx.astype(jnp.bfloat16), w1b,
                preferred_element_type=jnp.float32)
    var = q - u * u
    scale = g1_ref[...] * lax.rsqrt(var + EPS)
    w1p_ref[...] = (w1 * scale).astype(jnp.bfloat16)
    b1p_ref[...] = be1_ref[...] - u * scale


def _stage_c(x_ref, w1p_ref, b1p_ref, w2_ref, b2_ref,
             h2_ref, s2_ref, s2sq_ref):
    i = pl.program_id(0)

    @pl.when(i == 0)
    def _():
        s2_ref[...] = jnp.zeros_like(s2_ref)
        s2sq_ref[...] = jnp.zeros_like(s2sq_ref)

    xb = x_ref[...].astype(jnp.bfloat16)
    h = jnp.dot(xb, w1p_ref[...], preferred_element_type=jnp.float32)
    h = _lrelu(h + b1p_ref[...])
    h2 = jnp.dot(h.astype(jnp.bfloat16), w2_ref[...],
                 preferred_element_type=jnp.float32) + b2_ref[...]
    h2_ref[...] = h2.astype(jnp.bfloat16)
    s2_ref[...] += jnp.sum(h2, axis=0, keepdims=True)
    s2sq_ref[...] += jnp.sum(h2 * h2, axis=0, keepdims=True)


def _stage_d(h2_ref, x_ref, s2_ref, s2sq_ref, g2_ref, be2_ref,
             w3_ref, b3_ref, w4_ref, b4_ref, w5_ref, b5_ref,
             w6_ref, b6_ref, xs2_ref):
    m2 = s2_ref[...] / N
    var2 = s2sq_ref[...] / N - m2 * m2
    scale2 = g2_ref[...] * lax.rsqrt(var2 + EPS)
    shift2 = be2_ref[...] - m2 * scale2
    h2 = _lrelu(h2_ref[...].astype(jnp.float32) * scale2 + shift2)
    h3 = _lrelu(jnp.dot(h2.astype(jnp.bfloat16), w3_ref[...],
                        preferred_element_type=jnp.float32) + b3_ref[...])
    h4 = _lrelu(jnp.dot(h3.astype(jnp.bfloat16), w4_ref[...],
                        preferred_element_type=jnp.float32) + b4_ref[...])
    h5 = _lrelu(jnp.dot(h4.astype(jnp.bfloat16), w5_ref[...],
                        preferred_element_type=jnp.float32) + b5_ref[...])
    logit = jnp.sum(h5 * w6_ref[...], axis=1, keepdims=True) + b6_ref[...]
    gate = jax.nn.sigmoid(logit)
    e = jnp.exp(gate)
    xe = x_ref[...] * e
    mask0 = lax.broadcasted_iota(jnp.int32, (TN, D2 - D), 1) == 0
    etail = jnp.where(mask0, e, 0.0)
    xs2_ref[...] = jnp.concatenate([xe, etail], axis=1)


def _sc_scatter(xs2_hbm, seg_hbm, sf_hbm, out_hbm, xbuf, segbuf, sfbuf):
    c = lax.axis_index("c")
    s = lax.axis_index("s")
    base_c = c * AROWS

    zrow = jnp.zeros((16,), jnp.float32)

    def zx(i, cr):
        xbuf[i // (D2 // 16), pl.ds((i % (D2 // 16)) * 16, 16)] = zrow
        return cr

    lax.fori_loop(0, BR * (D2 // 16), zx, 0)
    r0 = base_c + s * 136
    pltpu.sync_copy(xbuf, out_hbm.at[pl.ds(r0, BR)])
    pltpu.sync_copy(xbuf.at[pl.ds(0, 136 - BR)],
                    out_hbm.at[pl.ds(r0 + BR, 136 - BR)])
    plsc.subcore_barrier()

    pltpu.sync_copy(sf_hbm.at[pl.ds(c * 16, 16)], sfbuf)
    spl = jnp.take(sfbuf[...], jnp.full((16,), s, jnp.int32))

    w = c * 16 + s
    start = w * CHUNK
    cnt = jnp.minimum(N - start, CHUNK)
    nb = (cnt + BR - 1) // BR
    spill_row = base_c + NG + s
    trash_row = base_c + TRASH
    lanes16 = lax.iota(jnp.int32, 16)

    def blk(jb, cr):
        base = jnp.minimum(jb * BR, cnt - BR)
        dup = jb * BR - base
        rr = start + base
        pltpu.sync_copy(xs2_hbm.at[pl.ds(rr, BR)], xbuf)
        pltpu.sync_copy(seg_hbm.at[pl.ds(rr, BR)], segbuf)
        for kk in range(BR // 16):
            lane = lanes16 + (kk * 16)
            sgb = segbuf[pl.ds(kk * 16, 16)]
            red = jnp.where(sgb == spl, spill_row, sgb + base_c)
            red = jnp.where(lane < dup, trash_row, red)
            segbuf[pl.ds(kk * 16, 16)] = red
        pltpu.sync_copy(xbuf, out_hbm.at[segbuf], add=True)
        return cr

    lax.fori_loop(0, nb, blk, 0)


def _stage_f(encp_ref, sf_ref, out_ref):
    t = encp_ref[0:NG, :] + encp_ref[AROWS:AROWS + NG, :]
    sp0 = encp_ref[NG:NG + 16, :]
    sp1 = encp_ref[AROWS + NG:AROWS + NG + 16, :]
    spill = jnp.concatenate([sp0, sp1], axis=0)
    iota = lax.broadcasted_iota(jnp.int32, (NW, NG), 1).astype(jnp.float32)
    onehot = (sf_ref[...] == iota).astype(jnp.bfloat16)
    t = t + lax.dot_general(onehot, spill.astype(jnp.bfloat16),
                            (((0,), (0,)), ((), ())),
                            preferred_element_type=jnp.float32)
    enc = t[:, :D]
    den = t[:, D:D + 1]
    r = 1.0 / jnp.where(den == 0.0, 1.0, den)
    out_ref[...] = enc * r


def _tc_front(x, W1, b1, g1, be1, W2, b2, g2, be2,
              W3, b3, W4, b4, W5, b5, W6, b6):
    f32 = jnp.float32
    bf16 = jnp.bfloat16
    row = lambda v: v.reshape(1, -1).astype(f32)

    c_mat, sx = pl.pallas_call(
        _stage_a,
        grid=(NT,),
        in_specs=[pl.BlockSpec((TN, D), lambda i: (i, 0))],
        out_specs=[pl.BlockSpec((D, D), lambda i: (0, 0)),
                   pl.BlockSpec((1, D), lambda i: (0, 0))],
        out_shape=[jax.ShapeDtypeStruct((D, D), f32),
                   jax.ShapeDtypeStruct((1, D), f32)],
    )(x)

    w1p, b1p = pl.pallas_call(
        _stage_p1,
        out_shape=[jax.ShapeDtypeStruct((D, H1), bf16),
                   jax.ShapeDtypeStruct((1, H1), f32)],
    )(c_mat, sx, W1, row(b1), row(g1), row(be1))

    h2pre, s2, s2sq = pl.pallas_call(
        _stage_c,
        grid=(NT,),
        in_specs=[pl.BlockSpec((TN, D), lambda i: (i, 0)),
                  pl.BlockSpec((D, H1), lambda i: (0, 0)),
                  pl.BlockSpec((1, H1), lambda i: (0, 0)),
                  pl.BlockSpec((H1, H2), lambda i: (0, 0)),
                  pl.BlockSpec((1, H2), lambda i: (0, 0))],
        out_specs=[pl.BlockSpec((TN, H2), lambda i: (i, 0)),
                   pl.BlockSpec((1, H2), lambda i: (0, 0)),
                   pl.BlockSpec((1, H2), lambda i: (0, 0))],
        out_shape=[jax.ShapeDtypeStruct((N, H2), jnp.bfloat16),
                   jax.ShapeDtypeStruct((1, H2), f32),
                   jax.ShapeDtypeStruct((1, H2), f32)],
    )(x, w1p, b1p, W2.astype(bf16), row(b2))

    xs2 = pl.pallas_call(
        _stage_d,
        grid=(NT,),
        in_specs=[pl.BlockSpec((TN, H2), lambda i: (i, 0)),
                  pl.BlockSpec((TN, D), lambda i: (i, 0)),
                  pl.BlockSpec((1, H2), lambda i: (0, 0)),
                  pl.BlockSpec((1, H2), lambda i: (0, 0)),
                  pl.BlockSpec((1, H2), lambda i: (0, 0)),
                  pl.BlockSpec((1, H2), lambda i: (0, 0)),
                  pl.BlockSpec((H2, 128), lambda i: (0, 0)),
                  pl.BlockSpec((1, 128), lambda i: (0, 0)),
                  pl.BlockSpec((128, 128), lambda i: (0, 0)),
                  pl.BlockSpec((1, 128), lambda i: (0, 0)),
                  pl.BlockSpec((128, 128), lambda i: (0, 0)),
                  pl.BlockSpec((1, 128), lambda i: (0, 0)),
                  pl.BlockSpec((1, 128), lambda i: (0, 0)),
                  pl.BlockSpec((1, 1), lambda i: (0, 0))],
        out_specs=[pl.BlockSpec((TN, D2), lambda i: (i, 0))],
        out_shape=[jax.ShapeDtypeStruct((N, D2), f32)],
    )(h2pre, x, s2, s2sq, row(g2), row(be2),
      W3.astype(bf16), row(b3), W4.astype(bf16), row(b4),
      W5.astype(bf16), row(b5), W6.reshape(1, 128).astype(f32),
      b6.reshape(1, 1).astype(f32))[0]
    return xs2


def _sc_call(xs2, seg, segfirst):
    f32 = jnp.float32
    mesh = plsc.VectorSubcoreMesh(core_axis_name="c", subcore_axis_name="s")
    encp = pl.kernel(
        _sc_scatter,
        out_type=jax.ShapeDtypeStruct((2 * AROWS, D2), f32),
        mesh=mesh,
        scratch_types=[pltpu.VMEM((BR, D2), f32),
                       pltpu.VMEM((BR,), jnp.int32),
                       pltpu.VMEM((16,), jnp.int32)],
    )(xs2, seg, segfirst)
    return encp


def _finalize(encp, sfcol):
    f32 = jnp.float32
    out = pl.pallas_call(
        _stage_f,
        in_specs=[pl.BlockSpec((2 * AROWS, D2), lambda: (0, 0)),
                  pl.BlockSpec((NW, 1), lambda: (0, 0))],
        out_specs=pl.BlockSpec((NG, D), lambda: (0, 0)),
        out_shape=jax.ShapeDtypeStruct((NG, D), f32),
    )(encp, sfcol)
    return out


def _stage_z(xs2_ref, segf_ref, out_ref):
    i = pl.program_id(0)

    @pl.when(i == 0)
    def _():
        out_ref[...] = jnp.zeros_like(out_ref)

    iota = lax.broadcasted_iota(jnp.int32, (TN, NG), 1).astype(jnp.float32)
    onehot = (segf_ref[...] == iota).astype(jnp.bfloat16)
    out_ref[...] += lax.dot_general(
        onehot, xs2_ref[...].astype(jnp.bfloat16),
        (((0,), (0,)), ((), ())), preferred_element_type=jnp.float32)


def _stage_zf(acc_ref, out_ref):
    t = acc_ref[...]
    enc = t[:, :D]
    den = t[:, D:D + 1]
    r = 1.0 / jnp.where(den == 0.0, 1.0, den)
    out_ref[...] = enc * r


def _tc_pool(xs2, segf):
    f32 = jnp.float32
    acc = pl.pallas_call(
        _stage_z,
        grid=(NT,),
        in_specs=[pl.BlockSpec((TN, D2), lambda i: (i, 0)),
                  pl.BlockSpec((TN, 1), lambda i: (i, 0))],
        out_specs=pl.BlockSpec((NG, D2), lambda i: (0, 0)),
        out_shape=jax.ShapeDtypeStruct((NG, D2), f32),
    )(xs2, segf)
    return pl.pallas_call(
        _stage_zf,
        in_specs=[pl.BlockSpec((NG, D2), lambda: (0, 0))],
        out_specs=pl.BlockSpec((NG, D), lambda: (0, 0)),
        out_shape=jax.ShapeDtypeStruct((NG, D), f32),
    )(acc)


def kernel(x, segment_ids, W1, b1, g1, be1, W2, b2, g2, be2,
           W3, b3, W4, b4, W5, b5, W6, b6):
    xs2 = _tc_front(x, W1, b1, g1, be1, W2, b2, g2, be2,
                    W3, b3, W4, b4, W5, b5, W6, b6)
    segf = segment_ids.astype(jnp.float32).reshape(N, 1)
    return _tc_pool(xs2, segf)
